# Initial kernel scaffold; baseline (speedup 1.0000x reference)
import functools

import jax
import jax.numpy as jnp
from jax import lax
from jax.experimental import pallas as pl
from jax.experimental.pallas import tpu as pltpu
from jax.experimental.pallas import tpu_sc as plsc

N = 10000
E = 320000
D = 128
G = 64
OUT = 10

NC = 2
NS = 16
NW = NC * NS

NP = 10240
ROWS_PER_TILE = NP // NS
CHUNK = 128
EDGES_PER_TILE = 10112
EP = EDGES_PER_TILE * NW
N_CHUNKS = EDGES_PER_TILE // CHUNK
COL_SENTINEL = N

_mesh = plsc.VectorSubcoreMesh(core_axis_name="c", subcore_axis_name="s")


def _worker_id():
    return lax.axis_index("s") * NC + lax.axis_index("c")


def _fill_buf_2d(buf, rows, width, value):
    v16 = jnp.full((16,), value, jnp.float32)

    def body(r, _):
        for j in range(width // 16):
            buf[r, pl.ds(j * 16, 16)] = v16
        return 0

    lax.fori_loop(0, rows, body, 0)


@functools.partial(
    pl.kernel,
    out_type=jax.ShapeDtypeStruct((NC, NP, 16), jnp.float32),
    mesh=_mesh,
    scratch_types=[
        pltpu.VMEM((CHUNK,), jnp.int32),
        pltpu.VMEM((CHUNK, 16), jnp.float32),
        pltpu.VMEM_SHARED((NP, 16), jnp.float32),
    ],
)
def _deg_kernel(col_hbm, out_hbm, colv, obuf, hist_sh):
    cid = lax.axis_index("c")
    tid = lax.axis_index("s")
    wid = _worker_id()

    _fill_buf_2d(obuf, CHUNK, 16, 0.0)
    for k in range(ROWS_PER_TILE // CHUNK):
        pltpu.sync_copy(obuf, hist_sh.at[pl.ds(tid * ROWS_PER_TILE + k * CHUNK, CHUNK)])
    _fill_buf_2d(obuf, CHUNK, 16, 1.0)
    plsc.subcore_barrier()

    base = wid * EDGES_PER_TILE

    def body(i, _):
        pltpu.sync_copy(col_hbm.at[pl.ds(base + i * CHUNK, CHUNK)], colv)
        pltpu.sync_copy(obuf, hist_sh.at[colv], add=True)
        return 0

    lax.fori_loop(0, N_CHUNKS, body, 0)
    plsc.subcore_barrier()

    pltpu.sync_copy(
        hist_sh.at[pl.ds(tid * ROWS_PER_TILE, ROWS_PER_TILE)],
        out_hbm.at[cid, pl.ds(tid * ROWS_PER_TILE, ROWS_PER_TILE)],
    )


@functools.partial(
    pl.kernel,
    out_type=jax.ShapeDtypeStruct((NC, NP, D), jnp.float32),
    mesh=_mesh,
    scratch_types=[
        pltpu.VMEM((CHUNK,), jnp.int32),
        pltpu.VMEM((CHUNK,), jnp.int32),
        pltpu.VMEM((CHUNK, D), jnp.float32),
        pltpu.VMEM_SHARED((NP, D), jnp.float32),
        pltpu.SemaphoreType.DMA,
    ],
)
def _edge_kernel(y_hbm, row_hbm, col_hbm, out_hbm, rowv, colv, gbuf, agg_sh, sem):
    cid = lax.axis_index("c")
    tid = lax.axis_index("s")
    wid = _worker_id()

    _fill_buf_2d(gbuf, CHUNK, D, 0.0)
    for k in range(ROWS_PER_TILE // CHUNK):
        pltpu.sync_copy(gbuf, agg_sh.at[pl.ds(tid * ROWS_PER_TILE + k * CHUNK, CHUNK)])
    plsc.subcore_barrier()

    base = wid * EDGES_PER_TILE

    def body(i, _):
        off = base + i * CHUNK
        pltpu.sync_copy(row_hbm.at[pl.ds(off, CHUNK)], rowv)
        pltpu.sync_copy(col_hbm.at[pl.ds(off, CHUNK)], colv)
        pltpu.async_copy(y_hbm.at[rowv], gbuf, sem).wait()
        pltpu.sync_copy(gbuf, agg_sh.at[colv], add=True)
        return 0

    lax.fori_loop(0, N_CHUNKS, body, 0)
    plsc.subcore_barrier()

    pltpu.sync_copy(
        agg_sh.at[pl.ds(tid * ROWS_PER_TILE, ROWS_PER_TILE)],
        out_hbm.at[cid, pl.ds(tid * ROWS_PER_TILE, ROWS_PER_TILE)],
    )


_BLK = 1280
_GRID = NP // _BLK


def _tc1_body(degp_ref, x_ref, w1_ref, y_ref, dis_ref):
    deg = degp_ref[0, :, 0:1] + degp_ref[1, :, 0:1]
    dis = jnp.where(deg > 0.0, lax.rsqrt(deg), 0.0)
    xl = lax.dot_general(x_ref[...], w1_ref[...], (((1,), (1,)), ((), ())),
                         preferred_element_type=jnp.float32)
    y_ref[...] = dis * xl
    dis_ref[...] = dis


def _tc1(degp, x_p, w1):
    return pl.pallas_call(
        _tc1_body,
        grid=(_GRID,),
        in_specs=[
            pl.BlockSpec((NC, _BLK, 16), lambda i: (0, i, 0)),
            pl.BlockSpec((_BLK, D), lambda i: (i, 0)),
            pl.BlockSpec((D, D), lambda i: (0, 0)),
        ],
        out_specs=[
            pl.BlockSpec((_BLK, D), lambda i: (i, 0)),
            pl.BlockSpec((_BLK, 1), lambda i: (i, 0)),
        ],
        out_shape=[
            jax.ShapeDtypeStruct((NP, D), jnp.float32),
            jax.ShapeDtypeStruct((NP, 1), jnp.float32),
        ],
    )(degp, x_p, w1)


def _tc2_body(aggp_ref, dis_ref, b1_ref, w2_ref, y2_ref):
    dis = dis_ref[...]
    h = dis * (aggp_ref[0] + aggp_ref[1]) + b1_ref[...]
    h = jnp.maximum(h, 0.0)
    y2 = lax.dot_general(h, w2_ref[...], (((1,), (1,)), ((), ())),
                         preferred_element_type=jnp.float32)
    y2_ref[...] = dis * y2


def _tc2(aggp, dis, b1, w2):
    return pl.pallas_call(
        _tc2_body,
        grid=(_GRID,),
        in_specs=[
            pl.BlockSpec((NC, _BLK, D), lambda i: (0, i, 0)),
            pl.BlockSpec((_BLK, 1), lambda i: (i, 0)),
            pl.BlockSpec((1, D), lambda i: (0, 0)),
            pl.BlockSpec((D, D), lambda i: (0, 0)),
        ],
        out_specs=pl.BlockSpec((_BLK, D), lambda i: (i, 0)),
        out_shape=jax.ShapeDtypeStruct((NP, D), jnp.float32),
    )(aggp, dis, b1, w2)


def _tc3_body(aggp_ref, dis_ref, b2_ref, batch_ref, wout_ref, bout_ref,
              logits_ref, sums_ref, cnt_ref):
    i = pl.program_id(0)

    @pl.when(i == 0)
    def _():
        sums_ref[...] = jnp.zeros_like(sums_ref)
        cnt_ref[...] = jnp.zeros_like(cnt_ref)

    dis = dis_ref[...]
    h = dis * (aggp_ref[0] + aggp_ref[1]) + b2_ref[...]
    b = batch_ref[...]
    gids = lax.broadcasted_iota(jnp.int32, (_BLK, G), 1)
    p = (b == gids).astype(jnp.float32)
    sums_ref[...] += lax.dot_general(p, h, (((0,), (0,)), ((), ())),
                                     preferred_element_type=jnp.float32)
    cnt_ref[...] += lax.dot_general(p, jnp.ones((_BLK, 1), jnp.float32),
                                    (((0,), (0,)), ((), ())),
                                    preferred_element_type=jnp.float32)

    @pl.when(i == _GRID - 1)
    def _():
        mean = sums_ref[...] / jnp.maximum(cnt_ref[...], 1.0)
        logits_ref[...] = lax.dot_general(
            mean, wout_ref[...], (((1,), (1,)), ((), ())),
            preferred_element_type=jnp.float32) + bout_ref[...]


def _tc3(aggp, dis, b2, batch_p, wout, bout):
    return pl.pallas_call(
        _tc3_body,
        grid=(_GRID,),
        in_specs=[
            pl.BlockSpec((NC, _BLK, D), lambda i: (0, i, 0)),
            pl.BlockSpec((_BLK, 1), lambda i: (i, 0)),
            pl.BlockSpec((1, D), lambda i: (0, 0)),
            pl.BlockSpec((_BLK, 1), lambda i: (i, 0)),
            pl.BlockSpec((OUT, D), lambda i: (0, 0)),
            pl.BlockSpec((1, OUT), lambda i: (0, 0)),
        ],
        out_specs=pl.BlockSpec((G, OUT), lambda i: (0, 0)),
        out_shape=jax.ShapeDtypeStruct((G, OUT), jnp.float32),
        scratch_shapes=[
            pltpu.VMEM((G, D), jnp.float32),
            pltpu.VMEM((G, 1), jnp.float32),
        ],
    )(aggp, dis, b2, batch_p, wout, bout)


def kernel(x, edge_index, batch, W1, b1, W2, b2, Wout, bout):
    row = edge_index[0]
    col = edge_index[1]
    row_p = jnp.concatenate([row, jnp.zeros((EP - E,), jnp.int32)])
    col_p = jnp.concatenate([col, jnp.full((EP - E,), COL_SENTINEL, jnp.int32)])
    x_p = jnp.concatenate([x, jnp.zeros((NP - N, D), jnp.float32)])
    batch_p = jnp.concatenate([batch, jnp.full((NP - N,), G, jnp.int32)])
    batch_p = batch_p.reshape(NP, 1)

    degp = _deg_kernel(col_p)
    y1, dis = _tc1(degp, x_p, W1)
    aggp1 = _edge_kernel(y1, row_p, col_p)
    y2 = _tc2(aggp1, dis, b1.reshape(1, D), W2)
    aggp2 = _edge_kernel(y2, row_p, col_p)
    logits = _tc3(aggp2, dis, b2.reshape(1, D), batch_p, Wout, bout.reshape(1, OUT))
    return logits


# speedup vs baseline: 9.4493x; 9.4493x over previous
import functools

import jax
import jax.numpy as jnp
from jax import lax
from jax.experimental import pallas as pl
from jax.experimental.pallas import tpu as pltpu
from jax.experimental.pallas import tpu_sc as plsc

N = 10000
E = 320000
D = 128
G = 64
OUT = 10

NC = 2
NS = 16
NW = NC * NS

NP = 10240
ROWS_PER_TILE = NP // NS
CHUNK = 128
EDGES_PER_TILE = 10112
EP = EDGES_PER_TILE * NW
N_CHUNKS = EDGES_PER_TILE // CHUNK
COL_SENTINEL = N

_mesh = plsc.VectorSubcoreMesh(core_axis_name="c", subcore_axis_name="s")


def _worker_id():
    return lax.axis_index("s") * NC + lax.axis_index("c")


def _fill_buf_2d(buf, rows, width, value):
    v16 = jnp.full((16,), value, jnp.float32)

    def body(r, _):
        for j in range(width // 16):
            buf[r, pl.ds(j * 16, 16)] = v16
        return 0

    lax.fori_loop(0, rows, body, 0)


def _deg_body(col_hbm, out_hbm, colv, obuf, hist_sh):
    cid = lax.axis_index("c")
    tid = lax.axis_index("s")
    wid = _worker_id()

    _fill_buf_2d(obuf, CHUNK, D, 0.0)
    for k in range(ROWS_PER_TILE // CHUNK):
        pltpu.sync_copy(obuf, hist_sh.at[pl.ds(tid * ROWS_PER_TILE + k * CHUNK, CHUNK)])
    _fill_buf_2d(obuf, CHUNK, D, 1.0)
    plsc.subcore_barrier()

    base = wid * EDGES_PER_TILE

    def body(i, _):
        pltpu.sync_copy(col_hbm.at[pl.ds(base + i * CHUNK, CHUNK)], colv)
        pltpu.sync_copy(obuf, hist_sh.at[colv], add=True)
        return 0

    lax.fori_loop(0, N_CHUNKS, body, 0)
    plsc.subcore_barrier()

    pltpu.sync_copy(
        hist_sh.at[pl.ds(tid * ROWS_PER_TILE, ROWS_PER_TILE)],
        out_hbm.at[cid, pl.ds(tid * ROWS_PER_TILE, ROWS_PER_TILE)],
    )


def _edge_body(y_hbm, row_hbm, col_hbm, out_hbm, rowv, colv, gbuf, agg_sh, sem):
    cid = lax.axis_index("c")
    tid = lax.axis_index("s")
    wid = _worker_id()

    _fill_buf_2d(gbuf, CHUNK, D, 0.0)
    for k in range(ROWS_PER_TILE // CHUNK):
        pltpu.sync_copy(gbuf, agg_sh.at[pl.ds(tid * ROWS_PER_TILE + k * CHUNK, CHUNK)])
    plsc.subcore_barrier()

    base = wid * EDGES_PER_TILE

    def body(i, _):
        off = base + i * CHUNK
        pltpu.sync_copy(row_hbm.at[pl.ds(off, CHUNK)], rowv)
        pltpu.sync_copy(col_hbm.at[pl.ds(off, CHUNK)], colv)
        pltpu.async_copy(y_hbm.at[rowv], gbuf, sem).wait()
        pltpu.sync_copy(gbuf, agg_sh.at[colv], add=True)
        return 0

    lax.fori_loop(0, N_CHUNKS, body, 0)
    plsc.subcore_barrier()

    pltpu.sync_copy(
        agg_sh.at[pl.ds(tid * ROWS_PER_TILE, ROWS_PER_TILE)],
        out_hbm.at[cid, pl.ds(tid * ROWS_PER_TILE, ROWS_PER_TILE)],
    )


def _make_deg_kernel(interpret=False):
    return pl.kernel(
        _deg_body,
        out_type=jax.ShapeDtypeStruct((NC, NP, D), jnp.float32),
        mesh=_mesh,
        scratch_types=[
            pltpu.VMEM((CHUNK,), jnp.int32),
            pltpu.VMEM((CHUNK, D), jnp.float32),
            pltpu.VMEM_SHARED((NP, D), jnp.float32),
        ],
        interpret=interpret,
    )


def _make_edge_kernel(interpret=False):
    return pl.kernel(
        _edge_body,
        out_type=jax.ShapeDtypeStruct((NC, NP, D), jnp.float32),
        mesh=_mesh,
        scratch_types=[
            pltpu.VMEM((CHUNK,), jnp.int32),
            pltpu.VMEM((CHUNK,), jnp.int32),
            pltpu.VMEM((CHUNK, D), jnp.float32),
            pltpu.VMEM_SHARED((NP, D), jnp.float32),
            pltpu.SemaphoreType.DMA,
        ],
        interpret=interpret,
    )


_deg_kernel = _make_deg_kernel()
_edge_kernel = _make_edge_kernel()


_BLK = 1280
_GRID = NP // _BLK


def _tc1_body(degp_ref, x_ref, w1_ref, y_ref, dis_ref):
    deg = degp_ref[0, :, 0:1] + degp_ref[1, :, 0:1]
    dis = jnp.where(deg > 0.0, lax.rsqrt(deg), 0.0)
    xl = lax.dot_general(x_ref[...], w1_ref[...], (((1,), (1,)), ((), ())),
                         preferred_element_type=jnp.float32)
    y_ref[...] = dis * xl
    dis_ref[...] = dis


def _tc1(degp, x_p, w1):
    return pl.pallas_call(
        _tc1_body,
        grid=(_GRID,),
        in_specs=[
            pl.BlockSpec((NC, _BLK, D), lambda i: (0, i, 0)),
            pl.BlockSpec((_BLK, D), lambda i: (i, 0)),
            pl.BlockSpec((D, D), lambda i: (0, 0)),
        ],
        out_specs=[
            pl.BlockSpec((_BLK, D), lambda i: (i, 0)),
            pl.BlockSpec((_BLK, 1), lambda i: (i, 0)),
        ],
        out_shape=[
            jax.ShapeDtypeStruct((NP, D), jnp.float32),
            jax.ShapeDtypeStruct((NP, 1), jnp.float32),
        ],
    )(degp, x_p, w1)


def _tc2_body(aggp_ref, dis_ref, b1_ref, w2_ref, y2_ref):
    dis = dis_ref[...]
    h = dis * (aggp_ref[0] + aggp_ref[1]) + b1_ref[...]
    h = jnp.maximum(h, 0.0)
    y2 = lax.dot_general(h, w2_ref[...], (((1,), (1,)), ((), ())),
                         preferred_element_type=jnp.float32)
    y2_ref[...] = dis * y2


def _tc2(aggp, dis, b1, w2):
    return pl.pallas_call(
        _tc2_body,
        grid=(_GRID,),
        in_specs=[
            pl.BlockSpec((NC, _BLK, D), lambda i: (0, i, 0)),
            pl.BlockSpec((_BLK, 1), lambda i: (i, 0)),
            pl.BlockSpec((1, D), lambda i: (0, 0)),
            pl.BlockSpec((D, D), lambda i: (0, 0)),
        ],
        out_specs=pl.BlockSpec((_BLK, D), lambda i: (i, 0)),
        out_shape=jax.ShapeDtypeStruct((NP, D), jnp.float32),
    )(aggp, dis, b1, w2)


def _tc3_body(aggp_ref, dis_ref, b2_ref, batch_ref, wout_ref, bout_ref,
              logits_ref, sums_ref, cnt_ref):
    i = pl.program_id(0)

    @pl.when(i == 0)
    def _():
        sums_ref[...] = jnp.zeros_like(sums_ref)
        cnt_ref[...] = jnp.zeros_like(cnt_ref)

    dis = dis_ref[...]
    h = dis * (aggp_ref[0] + aggp_ref[1]) + b2_ref[...]
    b = batch_ref[...]
    gids = lax.broadcasted_iota(jnp.int32, (_BLK, G), 1)
    p = (b == gids).astype(jnp.float32)
    sums_ref[...] += lax.dot_general(p, h, (((0,), (0,)), ((), ())),
                                     preferred_element_type=jnp.float32)
    cnt_ref[...] += lax.dot_general(p, jnp.ones((_BLK, 1), jnp.float32),
                                    (((0,), (0,)), ((), ())),
                                    preferred_element_type=jnp.float32)

    @pl.when(i == _GRID - 1)
    def _():
        mean = sums_ref[...] / jnp.maximum(cnt_ref[...], 1.0)
        logits_ref[...] = lax.dot_general(
            mean, wout_ref[...], (((1,), (1,)), ((), ())),
            preferred_element_type=jnp.float32) + bout_ref[...]


def _tc3(aggp, dis, b2, batch_p, wout, bout):
    return pl.pallas_call(
        _tc3_body,
        grid=(_GRID,),
        in_specs=[
            pl.BlockSpec((NC, _BLK, D), lambda i: (0, i, 0)),
            pl.BlockSpec((_BLK, 1), lambda i: (i, 0)),
            pl.BlockSpec((1, D), lambda i: (0, 0)),
            pl.BlockSpec((_BLK, 1), lambda i: (i, 0)),
            pl.BlockSpec((OUT, D), lambda i: (0, 0)),
            pl.BlockSpec((1, OUT), lambda i: (0, 0)),
        ],
        out_specs=pl.BlockSpec((G, OUT), lambda i: (0, 0)),
        out_shape=jax.ShapeDtypeStruct((G, OUT), jnp.float32),
        scratch_shapes=[
            pltpu.VMEM((G, D), jnp.float32),
            pltpu.VMEM((G, 1), jnp.float32),
        ],
    )(aggp, dis, b2, batch_p, wout, bout)


def kernel(x, edge_index, batch, W1, b1, W2, b2, Wout, bout):
    row = edge_index[0]
    col = edge_index[1]
    row_p = jnp.concatenate([row, jnp.zeros((EP - E,), jnp.int32)])
    col_p = jnp.concatenate([col, jnp.full((EP - E,), COL_SENTINEL, jnp.int32)])
    x_p = jnp.concatenate([x, jnp.zeros((NP - N, D), jnp.float32)])
    batch_p = jnp.concatenate([batch, jnp.full((NP - N,), G, jnp.int32)])
    batch_p = batch_p.reshape(NP, 1)

    degp = _deg_kernel(col_p)
    y1, dis = _tc1(degp, x_p, W1)
    aggp1 = _edge_kernel(y1, row_p, col_p)
    y2 = _tc2(aggp1, dis, b1.reshape(1, D), W2)
    aggp2 = _edge_kernel(y2, row_p, col_p)
    logits = _tc3(aggp2, dis, b2.reshape(1, D), batch_p, Wout, bout.reshape(1, OUT))
    return logits
